# baseline (speedup 1.0000x reference)
import functools

import jax
import jax.numpy as jnp
from jax import lax
from jax.experimental import pallas as pl
from jax.experimental.pallas import tpu as pltpu
from jax.experimental.pallas import tpu_sc as plsc

N = 10000
E = 640000
D = 128
NSUB = 16
N_PAD = 10240
ROWS_PER_SUB = N_PAD // NSUB
CPAD = ROWS_PER_SUB
EDGES_PER_SUB = E // NSUB
CHUNK = 80
NCHUNK = EDGES_PER_SUB // CHUNK

BLK = 2000


def _proj_body(x_ref, w_ref, b_ref, o_ref):
    h = jnp.dot(x_ref[...], w_ref[...], preferred_element_type=jnp.float32)
    o_ref[...] = h + b_ref[...]


def _proj(x, w, b):
    return pl.pallas_call(
        _proj_body,
        grid=(N // BLK,),
        in_specs=[
            pl.BlockSpec((BLK, D), lambda i: (i, 0)),
            pl.BlockSpec((D, D), lambda i: (0, 0)),
            pl.BlockSpec((1, D), lambda i: (0, 0)),
        ],
        out_specs=pl.BlockSpec((BLK, D), lambda i: (i, 0)),
        out_shape=jax.ShapeDtypeStruct((N, D), jnp.float32),
    )(x, w, b.reshape(1, D))


def _update_body(a_ref, c_ref, h_ref, wl_ref, b_ref, wr_ref, o_ref):
    mean = a_ref[...] / jnp.maximum(c_ref[...], 1.0)
    r = jnp.dot(mean, wl_ref[...], preferred_element_type=jnp.float32)
    r = r + b_ref[...]
    r = r + jnp.dot(h_ref[...], wr_ref[...],
                    preferred_element_type=jnp.float32)
    o_ref[...] = jnp.maximum(r, 0.0)


def _update(aggbuf, cnt, h_tab, wl, bl, wr):
    return pl.pallas_call(
        _update_body,
        grid=(N // BLK,),
        in_specs=[
            pl.BlockSpec((BLK, D), lambda i: (i, 0)),
            pl.BlockSpec((BLK, 1), lambda i: (i, 0)),
            pl.BlockSpec((BLK, D), lambda i: (i, 0)),
            pl.BlockSpec((D, D), lambda i: (0, 0)),
            pl.BlockSpec((1, D), lambda i: (0, 0)),
            pl.BlockSpec((D, D), lambda i: (0, 0)),
        ],
        out_specs=pl.BlockSpec((BLK, D), lambda i: (i, 0)),
        out_shape=jax.ShapeDtypeStruct((N, D), jnp.float32),
    )(aggbuf, cnt, h_tab, wl, bl.reshape(1, D), wr)


def _update_final_body(a_ref, c_ref, h_ref, wl_ref, b_ref, wr_ref,
                       wh_ref, bh_ref, o_ref):
    mean = a_ref[...] / jnp.maximum(c_ref[...], 1.0)
    r = jnp.dot(mean, wl_ref[...], preferred_element_type=jnp.float32)
    r = r + b_ref[...]
    r = r + jnp.dot(h_ref[...], wr_ref[...],
                    preferred_element_type=jnp.float32)
    r = jnp.maximum(r, 0.0)
    r = jnp.dot(r, wh_ref[...], preferred_element_type=jnp.float32)
    o_ref[...] = r + bh_ref[...]


def _update_final(aggbuf, cnt, h_tab, wl, bl, wr, wh, bh):
    return pl.pallas_call(
        _update_final_body,
        grid=(N // BLK,),
        in_specs=[
            pl.BlockSpec((BLK, D), lambda i: (i, 0)),
            pl.BlockSpec((BLK, 1), lambda i: (i, 0)),
            pl.BlockSpec((BLK, D), lambda i: (i, 0)),
            pl.BlockSpec((D, D), lambda i: (0, 0)),
            pl.BlockSpec((1, D), lambda i: (0, 0)),
            pl.BlockSpec((D, D), lambda i: (0, 0)),
            pl.BlockSpec((D, D), lambda i: (0, 0)),
            pl.BlockSpec((1, D), lambda i: (0, 0)),
        ],
        out_specs=pl.BlockSpec((BLK, D), lambda i: (i, 0)),
        out_shape=jax.ShapeDtypeStruct((N, D), jnp.float32),
    )(aggbuf, cnt, h_tab, wl, bl.reshape(1, D), wr, wh, bh.reshape(1, D))


def _make_agg(with_counts):
    mesh = plsc.VectorSubcoreMesh(core_axis_name="c", subcore_axis_name="s")

    out_type = [
        jax.ShapeDtypeStruct((N_PAD, D), jnp.float32),
        jax.ShapeDtypeStruct((N_PAD, D), jnp.float32),
    ]
    scratch = [
        pltpu.VMEM((CHUNK,), jnp.int32),
        pltpu.VMEM((CHUNK,), jnp.int32),
        pltpu.VMEM((CHUNK, D), jnp.float32),
        pltpu.VMEM_SHARED((N_PAD, D), jnp.float32),
        pltpu.SemaphoreType.DMA,
    ]
    if with_counts:
        out_type += [
            jax.ShapeDtypeStruct((N_PAD,), jnp.float32),
            jax.ShapeDtypeStruct((N_PAD,), jnp.float32),
        ]
        scratch += [
            pltpu.VMEM((N_PAD,), jnp.int32),
            pltpu.VMEM((NSUB * CPAD,), jnp.int32),
            pltpu.VMEM((CPAD,), jnp.float32),
            pltpu.VMEM_SHARED((NSUB * N_PAD,), jnp.int32),
        ]

    @functools.partial(
        pl.kernel,
        mesh=mesh,
        out_type=tuple(out_type),
        scratch_types=scratch,
        compiler_params=pltpu.CompilerParams(needs_layout_passes=False),
    )
    def agg(tab_u, tab_i, ei_u2i, ei_i2u, zeros_hbm, *refs):
        if with_counts:
            (out_i, out_u, cnt_i, cnt_u,
             src_idx, dst_idx, rows, acc, sem,
             hist, hbuf, cred, hist_sh) = refs
        else:
            (out_i, out_u, src_idx, dst_idx, rows, acc, sem) = refs

        c = lax.axis_index("c")
        s = lax.axis_index("s")
        r0 = s * ROWS_PER_SUB

        pltpu.sync_copy(zeros_hbm.at[pl.ds(r0, ROWS_PER_SUB)],
                        acc.at[pl.ds(r0, ROWS_PER_SUB)])
        if with_counts:
            zero16 = jnp.zeros((16,), jnp.int32)

            def zbody(i, carry):
                hist[pl.ds(i * 16, 16)] = zero16
                return carry

            lax.fori_loop(0, N_PAD // 16, zbody, 0)
        plsc.subcore_barrier()

        def run(tab, ei, out, cnt_out):
            e0 = s * EDGES_PER_SUB

            def body(k, carry):
                base = e0 + k * CHUNK
                pltpu.sync_copy(ei.at[pl.ds(base, CHUNK)], src_idx)
                pltpu.sync_copy(ei.at[pl.ds(E + base, CHUNK)], dst_idx)
                pltpu.async_copy(tab.at[src_idx], rows, sem).wait()
                pltpu.sync_copy(rows, acc.at[dst_idx], add=True)
                if with_counts:
                    for g in range(CHUNK // 16):
                        d = dst_idx[pl.ds(g * 16, 16)]
                        occ, last = plsc.scan_count(d)
                        plsc.addupdate_scatter(hist, [d], occ, mask=last)
                return carry

            lax.fori_loop(0, NCHUNK, body, 0)

            if with_counts:
                pltpu.sync_copy(hist, hist_sh.at[pl.ds(s * N_PAD, N_PAD)])
            plsc.subcore_barrier()
            pltpu.sync_copy(acc.at[pl.ds(r0, ROWS_PER_SUB)],
                            out.at[pl.ds(r0, ROWS_PER_SUB)])
            if with_counts:
                for t in range(NSUB):
                    pltpu.sync_copy(
                        hist_sh.at[pl.ds(t * N_PAD + r0, ROWS_PER_SUB)],
                        hbuf.at[pl.ds(t * CPAD, ROWS_PER_SUB)])

                def rbody(v, carry):
                    tot = hbuf[pl.ds(v * 16, 16)]
                    for t in range(1, NSUB):
                        tot = tot + hbuf[pl.ds(t * CPAD + v * 16, 16)]
                    cred[pl.ds(v * 16, 16)] = tot.astype(jnp.float32)
                    return carry

                lax.fori_loop(0, ROWS_PER_SUB // 16, rbody, 0)
                pltpu.sync_copy(cred.at[pl.ds(0, ROWS_PER_SUB)],
                                cnt_out.at[pl.ds(r0, ROWS_PER_SUB)])

        @pl.when(c == 0)
        def _():
            run(tab_u, ei_u2i, out_i, cnt_i if with_counts else None)

        @pl.when(c == 1)
        def _():
            run(tab_i, ei_i2u, out_u, cnt_u if with_counts else None)

    return agg


_agg0 = _make_agg(with_counts=True)
_agg1 = _make_agg(with_counts=False)


def kernel(x_user, x_item, edge_index_u2i, edge_index_i2u, W_pu, b_pu,
           W_pi, b_pi, Wl0_ui, bl0_ui, Wr0_ui, Wl0_iu, bl0_iu, Wr0_iu,
           Wl1_ui, bl1_ui, Wr1_ui, Wl1_iu, bl1_iu, Wr1_iu, W_hu, b_hu,
           W_hi, b_hi):
    zeros = jnp.zeros((N_PAD, D), jnp.float32)
    ei_u2i = edge_index_u2i.reshape(-1)
    ei_i2u = edge_index_i2u.reshape(-1)

    tab_u = _proj(x_user, W_pu, b_pu)
    tab_i = _proj(x_item, W_pi, b_pi)

    agg_i0, agg_u0, cnt_i, cnt_u = _agg0(tab_u, tab_i, ei_u2i, ei_i2u, zeros)
    cnt_i = cnt_i.reshape(N_PAD, 1)
    cnt_u = cnt_u.reshape(N_PAD, 1)
    tab_i = _update(agg_i0, cnt_i, tab_i, Wl0_ui, bl0_ui, Wr0_ui)
    tab_u = _update(agg_u0, cnt_u, tab_u, Wl0_iu, bl0_iu, Wr0_iu)

    agg_i1, agg_u1 = _agg1(tab_u, tab_i, ei_u2i, ei_i2u, zeros)
    emb_i = _update_final(agg_i1, cnt_i, tab_i, Wl1_ui, bl1_ui, Wr1_ui,
                          W_hi, b_hi)
    emb_u = _update_final(agg_u1, cnt_u, tab_u, Wl1_iu, bl1_iu, Wr1_iu,
                          W_hu, b_hu)
    return (emb_u, emb_i)
